# Initial kernel scaffold; baseline (speedup 1.0000x reference)
import jax
import jax.numpy as jnp
from jax.experimental import pallas as pl


def kernel(pred_probs, margins, edge_index, norm_weights, raw_weights, learnable_indices, frozen_indices, num_iter):
    raise NotImplementedError("write your pallas kernel here")


# speedup vs baseline: 23.2569x; 23.2569x over previous
import functools

import jax
import jax.numpy as jnp
from jax import lax
from jax.experimental import pallas as pl
from jax.experimental.pallas import tpu as pltpu
from jax.experimental.pallas import tpu_sc as plsc

ALPHA = 0.999

NC = 2
NS = 16
NW = NC * NS
LANES = 16
CHUNK = 128
BLKC = 8
EPB = CHUNK * BLKC


def _make_step(n: int, c: int, blocks_per_tile: int):
    rpt = (-(-n // NS) + 7) // 8 * 8
    assert n % 8 == 0 and n >= rpt

    mesh = plsc.VectorSubcoreMesh(core_axis_name="c", subcore_axis_name="s",
                                  num_cores=NC, num_subcores=NS)

    @functools.partial(
        pl.kernel,
        mesh=mesh,
        compiler_params=pltpu.CompilerParams(use_tc_tiling_on_sc=False),
        out_type=(
            jax.ShapeDtypeStruct((n, c), jnp.float32),
            jax.ShapeDtypeStruct((n, c), jnp.float32),
        ),
        scratch_types=[
            pltpu.VMEM_SHARED((n, c), jnp.float32),
            pltpu.VMEM((BLKC, CHUNK), jnp.int32),
            pltpu.VMEM((BLKC, CHUNK), jnp.int32),
            pltpu.VMEM((BLKC, CHUNK), jnp.float32),
            pltpu.VMEM((BLKC, CHUNK, LANES), jnp.float32),
            pltpu.VMEM((rpt, LANES), jnp.float32),
            pltpu.SemaphoreType.DMA,
            pltpu.SemaphoreType.DMA,
        ],
    )
    def step(z_hbm, col_hbm, row_hbm, w_hbm, acc0_hbm, acc1_hbm,
             acc_sp, colblk, rowblk, wblk, gat, zbuf, gsem, ssem):
        cid = lax.axis_index("c")
        sid = lax.axis_index("s")
        wid = sid * NC + cid

        zeros_row = jnp.zeros((LANES,), jnp.float32)

        @plsc.parallel_loop(0, rpt, unroll=4)
        def _zero(i):
            zbuf[i, :] = zeros_row

        base = jnp.minimum(sid * rpt, n - rpt)
        pltpu.sync_copy(zbuf, acc_sp.at[pl.ds(base, rpt)])
        plsc.subcore_barrier()

        blk0 = wid * blocks_per_tile * BLKC

        def blk_body(b, carry):
            rb = blk0 + b * BLKC
            pltpu.sync_copy(col_hbm.at[pl.ds(rb, BLKC), :], colblk)
            pltpu.sync_copy(row_hbm.at[pl.ds(rb, BLKC), :], rowblk)
            pltpu.sync_copy(w_hbm.at[pl.ds(rb, BLKC), :], wblk)
            descs = [
                pltpu.async_copy(z_hbm.at[colblk.at[j]], gat.at[j], gsem)
                for j in range(BLKC)
            ]
            for d in descs:
                d.wait()
            for j in range(BLKC):
                @plsc.parallel_loop(0, CHUNK // LANES, unroll=2)
                def _scale(g, j=j):
                    w16 = wblk[j, pl.ds(g * LANES, LANES)]
                    for u in range(LANES):
                        k = g * LANES + u
                        gat[j, k, :] = gat[j, k, :] * w16[u]
            descs2 = [
                pltpu.async_copy(gat.at[j], acc_sp.at[rowblk.at[j]], ssem,
                                 add=True)
                for j in range(BLKC)
            ]
            for d in descs2:
                d.wait()
            return carry

        lax.fori_loop(0, blocks_per_tile, blk_body, 0)
        plsc.subcore_barrier()

        @pl.when(cid == 0)
        def _():
            pltpu.sync_copy(acc_sp.at[pl.ds(base, rpt)],
                            acc0_hbm.at[pl.ds(base, rpt)])

        @pl.when(cid == 1)
        def _():
            pltpu.sync_copy(acc_sp.at[pl.ds(base, rpt)],
                            acc1_hbm.at[pl.ds(base, rpt)])

    return step


def kernel(pred_probs, margins, edge_index, norm_weights, raw_weights,
           learnable_indices, frozen_indices, num_iter):
    n, c = pred_probs.shape
    e = edge_index.shape[1]
    f32 = jnp.float32

    conf = jnp.zeros((n,), f32)
    conf = conf.at[frozen_indices].set(1.0)
    conf = conf.at[learnable_indices].set(jax.nn.sigmoid(raw_weights))
    preds = jnp.argmax(pred_probs, axis=1)
    pred_onehot = jax.nn.one_hot(preds, c, dtype=f32)
    inj = jnp.zeros((n,), f32)
    inj = inj.at[learnable_indices].set(1.0)
    inj = inj.at[frozen_indices].set(1.0)
    h = pred_onehot * (conf * margins)[:, None] * inj[:, None]

    blocks_per_tile = -(-e // (NW * EPB))
    epad = NW * blocks_per_tile * EPB
    pad = epad - e
    row = jnp.concatenate([edge_index[0], jnp.zeros((pad,), jnp.int32)])
    col = jnp.concatenate([edge_index[1], jnp.zeros((pad,), jnp.int32)])
    w = jnp.concatenate([norm_weights, jnp.zeros((pad,), f32)])
    nrows = epad // CHUNK
    row2d = row.reshape(nrows, CHUNK)
    col2d = col.reshape(nrows, CHUNK)
    w2d = w.reshape(nrows, CHUNK)

    step = _make_step(n, c, blocks_per_tile)

    def body(_, z):
        acc0, acc1 = step(z, col2d, row2d, w2d)
        return (1.0 - ALPHA) * h + ALPHA * (acc0 + acc1)

    return lax.fori_loop(0, num_iter, body, h)


# speedup vs baseline: 25.7383x; 1.1067x over previous
import jax
import jax.numpy as jnp
from jax import lax
from jax.experimental import pallas as pl
from jax.experimental.pallas import tpu as pltpu
from jax.experimental.pallas import tpu_sc as plsc

ALPHA = 0.999

NC = 2
NS = 16
NW = NC * NS
LANES = 16
CHUNK = 128
BLKC = 12
EPB = CHUNK * BLKC
ZROWS = 512


def _make_step(n: int, c: int, bpt: int):
    rpt = (-(-n // NS) + 7) // 8 * 8
    assert n % 8 == 0 and n >= rpt and bpt >= 2

    mesh = plsc.VectorSubcoreMesh(core_axis_name="c", subcore_axis_name="s",
                                  num_cores=NC, num_subcores=NS)

    def step_body(z_hbm, col_hbm, row_hbm, w_hbm, acc0_hbm, acc1_hbm,
                  acc_sp, colblk, rowblk, wblk, gat, zbuf, esem, gsem, ssem):
        cid = lax.axis_index("c")
        sid = lax.axis_index("s")
        wid = sid * NC + cid

        zeros_row = jnp.zeros((LANES,), jnp.float32)

        @plsc.parallel_loop(0, ZROWS, unroll=4)
        def _zero(i):
            zbuf[i, :] = zeros_row

        base = jnp.minimum(sid * rpt, n - rpt)
        nfull, rem = divmod(rpt, ZROWS)
        for r in range(nfull):
            pltpu.sync_copy(zbuf, acc_sp.at[pl.ds(base + r * ZROWS, ZROWS)])
        if rem:
            pltpu.sync_copy(zbuf.at[pl.ds(0, rem)],
                            acc_sp.at[pl.ds(base + nfull * ZROWS, rem)])
        plsc.subcore_barrier()

        row0 = wid * bpt * BLKC

        def fire_edges(b, buf):
            rb = row0 + b * BLKC
            pltpu.async_copy(col_hbm.at[pl.ds(rb, BLKC), :], colblk.at[buf],
                             esem)
            pltpu.async_copy(row_hbm.at[pl.ds(rb, BLKC), :], rowblk.at[buf],
                             esem)
            pltpu.async_copy(w_hbm.at[pl.ds(rb, BLKC), :], wblk.at[buf], esem)

        def wait_edges(buf):
            pltpu.make_async_copy(col_hbm.at[pl.ds(0, BLKC), :],
                                  colblk.at[buf], esem).wait()
            pltpu.make_async_copy(row_hbm.at[pl.ds(0, BLKC), :],
                                  rowblk.at[buf], esem).wait()
            pltpu.make_async_copy(w_hbm.at[pl.ds(0, BLKC), :],
                                  wblk.at[buf], esem).wait()

        def fire_gathers(buf):
            for j in range(BLKC):
                pltpu.async_copy(z_hbm.at[colblk.at[buf, j]], gat.at[buf, j],
                                 gsem)

        def wait_gathers(buf):
            for j in range(BLKC):
                pltpu.make_async_copy(z_hbm.at[pl.ds(0, CHUNK)],
                                      gat.at[buf, j], gsem).wait()

        def scale(buf):
            for j in range(BLKC):
                @plsc.parallel_loop(0, CHUNK // LANES, unroll=2)
                def _s(g, j=j):
                    w16 = wblk[buf, j, pl.ds(g * LANES, LANES)]
                    for u in range(LANES):
                        k = g * LANES + u
                        gat[buf, j, k, :] = gat[buf, j, k, :] * w16[u]

        def fire_scatters(buf):
            for j in range(BLKC):
                pltpu.async_copy(gat.at[buf, j], acc_sp.at[rowblk.at[buf, j]],
                                 ssem, add=True)

        def wait_scatters(buf):
            for j in range(BLKC):
                pltpu.make_async_copy(z_hbm.at[pl.ds(0, CHUNK)],
                                      gat.at[buf, j], ssem).wait()

        fire_edges(0, 0)
        wait_edges(0)
        fire_gathers(0)

        def loop_body(b, carry):
            buf = lax.rem(b, 2)
            oth = 1 - buf

            @pl.when(b > 0)
            def _():
                wait_scatters(oth)

            @pl.when(b + 1 < bpt)
            def _():
                fire_edges(b + 1, oth)

            wait_gathers(buf)
            scale(buf)
            fire_scatters(buf)

            @pl.when(b + 1 < bpt)
            def _():
                wait_edges(oth)
                fire_gathers(oth)

            return carry

        lax.fori_loop(0, bpt, loop_body, 0)
        wait_scatters(lax.rem(bpt - 1, 2))
        plsc.subcore_barrier()

        @pl.when(cid == 0)
        def _():
            pltpu.sync_copy(acc_sp.at[pl.ds(base, rpt)],
                            acc0_hbm.at[pl.ds(base, rpt)])

        @pl.when(cid == 1)
        def _():
            pltpu.sync_copy(acc_sp.at[pl.ds(base, rpt)],
                            acc1_hbm.at[pl.ds(base, rpt)])

    return pl.kernel(
        step_body,
        mesh=mesh,
        compiler_params=pltpu.CompilerParams(use_tc_tiling_on_sc=False),
        out_type=(
            jax.ShapeDtypeStruct((n, c), jnp.float32),
            jax.ShapeDtypeStruct((n, c), jnp.float32),
        ),
        scratch_types=[
            pltpu.VMEM_SHARED((n, c), jnp.float32),
            pltpu.VMEM((2, BLKC, CHUNK), jnp.int32),
            pltpu.VMEM((2, BLKC, CHUNK), jnp.int32),
            pltpu.VMEM((2, BLKC, CHUNK), jnp.float32),
            pltpu.VMEM((2, BLKC, CHUNK, LANES), jnp.float32),
            pltpu.VMEM((ZROWS, LANES), jnp.float32),
            pltpu.SemaphoreType.DMA,
            pltpu.SemaphoreType.DMA,
            pltpu.SemaphoreType.DMA,
        ],
    )


def kernel(pred_probs, margins, edge_index, norm_weights, raw_weights,
           learnable_indices, frozen_indices, num_iter):
    n, c = pred_probs.shape
    e = edge_index.shape[1]
    f32 = jnp.float32

    conf = jnp.zeros((n,), f32)
    conf = conf.at[frozen_indices].set(1.0)
    conf = conf.at[learnable_indices].set(jax.nn.sigmoid(raw_weights))
    preds = jnp.argmax(pred_probs, axis=1)
    pred_onehot = jax.nn.one_hot(preds, c, dtype=f32)
    inj = jnp.zeros((n,), f32)
    inj = inj.at[learnable_indices].set(1.0)
    inj = inj.at[frozen_indices].set(1.0)
    h = pred_onehot * (conf * margins)[:, None] * inj[:, None]

    bpt = -(-e // (NW * EPB))
    epad = NW * bpt * EPB
    pad = epad - e
    row = jnp.concatenate([edge_index[0], jnp.zeros((pad,), jnp.int32)])
    col = jnp.concatenate([edge_index[1], jnp.zeros((pad,), jnp.int32)])
    w = jnp.concatenate([norm_weights, jnp.zeros((pad,), f32)])
    nrows = epad // CHUNK
    row2d = row.reshape(nrows, CHUNK)
    col2d = col.reshape(nrows, CHUNK)
    w2d = w.reshape(nrows, CHUNK)

    step = _make_step(n, c, bpt)

    def body(_, z):
        acc0, acc1 = step(z, col2d, row2d, w2d)
        return (1.0 - ALPHA) * h + ALPHA * (acc0 + acc1)

    return lax.fori_loop(0, num_iter, body, h)
